# baseline (speedup 1.0000x reference)
import jax
import jax.numpy as jnp
from jax.experimental import pallas as pl
from jax.experimental.pallas import tpu as pltpu

ROWS, COLS = 8192, 4096
BLK = 1024
K = 16
FEAT = 64
PATCH = 16


def _sum_topk_kernel(mask_ref, sim_ref, avg_ref, idx_ref, scores_ref,
                     points_ref, acc_ref, cnt_ref):
    i = pl.program_id(0)

    @pl.when(i == 0)
    def _init():
        acc_ref[...] = jnp.zeros_like(acc_ref)
        cnt_ref[0, 0] = jnp.float32(0.0)

    m = mask_ref[...]
    blk = sim_ref[...]
    acc_ref[...] += jnp.dot(m, blk, preferred_element_type=jnp.float32)
    cnt_ref[0, 0] += jnp.sum(m)

    @pl.when(i == pl.num_programs(0) - 1)
    def _final():
        cnt = cnt_ref[0, 0]
        avg = acc_ref[...] / cnt
        avg_ref[...] = avg
        a = avg
        col = jax.lax.broadcasted_iota(jnp.int32, (1, COLS), 1)
        idxs = []
        scs = []
        for _ in range(K):
            mn = jnp.min(a)
            sel = jnp.where(a == mn, col, jnp.int32(COLS))
            ix = jnp.min(sel)
            a = jnp.where(col == ix, jnp.float32(jnp.inf), a)
            idxs.append(ix)
            scs.append(mn)
        idxv = jnp.stack(idxs)
        scv = jnp.stack(scs)
        idx_ref[0, :] = idxv
        scores_ref[0, :] = scv
        xf = (idxv % FEAT).astype(jnp.float32) * PATCH + (PATCH // 2)
        yf = (idxv // FEAT).astype(jnp.float32) * PATCH + (PATCH // 2)
        points_ref[0, :] = xf
        points_ref[1, :] = yf
        points_ref[2, :] = scv


def _run(mask2d, similarity_map, interpret=False):
    return pl.pallas_call(
        _sum_topk_kernel,
        grid=(ROWS // BLK,),
        in_specs=[
            pl.BlockSpec((1, BLK), lambda i: (0, i)),
            pl.BlockSpec((BLK, COLS), lambda i: (i, 0)),
        ],
        out_specs=[
            pl.BlockSpec((1, COLS), lambda i: (0, 0)),
            pl.BlockSpec((1, K), lambda i: (0, 0)),
            pl.BlockSpec((1, K), lambda i: (0, 0)),
            pl.BlockSpec((3, K), lambda i: (0, 0)),
        ],
        out_shape=[
            jax.ShapeDtypeStruct((1, COLS), jnp.float32),
            jax.ShapeDtypeStruct((1, K), jnp.int32),
            jax.ShapeDtypeStruct((1, K), jnp.float32),
            jax.ShapeDtypeStruct((3, K), jnp.float32),
        ],
        scratch_shapes=[
            pltpu.VMEM((1, COLS), jnp.float32),
            pltpu.SMEM((1, 1), jnp.float32),
        ],
        interpret=interpret,
    )(mask2d, similarity_map)


@jax.jit
def kernel(similarity_map, ref_mask):
    mask2d = ref_mask.astype(jnp.float32).reshape(1, ROWS)
    avg, idx, scores, points = _run(mask2d, similarity_map)
    return (avg.reshape(COLS), idx.reshape(K), scores.reshape(K), points.T)
